# baseline (speedup 1.0000x reference)
import functools

import jax
import jax.numpy as jnp
from jax import lax
from jax.experimental import pallas as pl
from jax.experimental.pallas import tpu as pltpu
from jax.experimental.pallas import tpu_sc as plsc

VOCAB = 100000
D = 128
B = 16384
NSLOT = 5
PCOL = 8

NC = 2
NS = 16
NW = NC * NS
BPW = B // NW
FLAT = BPW * NSLOT
SUB = BPW // 128
NROW = NSLOT * SUB
TC_ROWS = 20000


def _tc_project_body(g_ref, w_ref, p_ref):
    p_ref[...] = jnp.dot(
        g_ref[...].astype(jnp.bfloat16),
        w_ref[...].astype(jnp.bfloat16),
        preferred_element_type=jnp.float32,
    )


def _tc_project(g_feats, w_pad):
    grid = VOCAB // TC_ROWS
    return pl.pallas_call(
        _tc_project_body,
        grid=(grid,),
        in_specs=[
            pl.BlockSpec((TC_ROWS, D), lambda i: (i, 0)),
            pl.BlockSpec((D, PCOL), lambda i: (0, 0)),
        ],
        out_specs=pl.BlockSpec((TC_ROWS, PCOL), lambda i: (i, 0)),
        out_shape=jax.ShapeDtypeStruct((VOCAB, PCOL), jnp.float32),
    )(g_feats, w_pad)


def _sc_deinterleave(samples_flat):
    mesh = plsc.VectorSubcoreMesh(core_axis_name="c", subcore_axis_name="s")

    @functools.partial(
        pl.kernel,
        mesh=mesh,
        out_type=jax.ShapeDtypeStruct((B * NSLOT,), jnp.int32),
        scratch_types=[
            pltpu.VMEM((NROW, 128), jnp.int32),
            pltpu.VMEM((FLAT,), jnp.int32),
            pltpu.SemaphoreType.DMA,
        ],
    )
    def sc_k(sflat_hbm, out_hbm, cidx, svT, sem):
        wid = lax.axis_index("s") * NC + lax.axis_index("c")
        base5 = wid * FLAT
        io5 = lax.iota(jnp.int32, 16) * NSLOT
        handles = []
        for r in range(NROW):
            j, s = r // SUB, r % SUB
            for c in range(8):
                t = s * 8 + c
                cidx[r, pl.ds(c * 16, 16)] = io5 + (base5 + j + 80 * t)
            handles.append(
                pltpu.async_copy(
                    sflat_hbm.at[cidx.at[r]], svT.at[pl.ds(r * 128, 128)], sem
                )
            )
        for h in handles:
            h.wait()
        pltpu.sync_copy(svT, out_hbm.at[pl.ds(base5, FLAT)])

    return sc_k(samples_flat)


def _sc_gather(p_flat, svt_flat, bias16):
    mesh = plsc.VectorSubcoreMesh(core_axis_name="c", subcore_axis_name="s")

    @functools.partial(
        pl.kernel,
        mesh=mesh,
        out_type=jax.ShapeDtypeStruct((B,), jnp.float32),
        scratch_types=[
            pltpu.VMEM((FLAT,), jnp.int32),
            pltpu.VMEM((NROW, 128), jnp.int32),
            pltpu.VMEM((NROW, 128), jnp.float32),
            pltpu.VMEM((BPW,), jnp.float32),
            pltpu.VMEM((16,), jnp.float32),
            pltpu.SemaphoreType.DMA,
        ],
    )
    def sc_k(pflat_hbm, svt_hbm, bias_hbm, out_hbm, sv, fidx, gbuf, acc, bv, sem):
        wid = lax.axis_index("s") * NC + lax.axis_index("c")
        base = wid * BPW
        pltpu.sync_copy(svt_hbm.at[pl.ds(base * NSLOT, FLAT)], sv)
        pltpu.sync_copy(bias_hbm, bv)
        handles = []
        for r in range(NROW):
            j = r // SUB
            for c in range(8):
                ids = sv[pl.ds(r * 128 + c * 16, 16)]
                fidx[r, pl.ds(c * 16, 16)] = ids * PCOL + j
            handles.append(
                pltpu.async_copy(pflat_hbm.at[fidx.at[r]], gbuf.at[r], sem)
            )
        for h in handles:
            h.wait()
        bias_v = bv[...]
        for c in range(BPW // 16):
            s, off = c // 8, (c % 8) * 16
            tot = bias_v
            for j in range(NSLOT):
                tot = tot + gbuf[j * SUB + s, pl.ds(off, 16)]
            acc[pl.ds(c * 16, 16)] = tot
        pltpu.sync_copy(acc, out_hbm.at[pl.ds(base, BPW)])

    return sc_k(p_flat, svt_flat, bias16)


def kernel(g_feats, samples, W, b):
    w_pad = jnp.zeros((D, PCOL), jnp.float32).at[:, :NSLOT].set(
        W.reshape(NSLOT, D).T
    )
    svt_flat = _sc_deinterleave(samples.reshape(-1))
    p = _tc_project(g_feats, w_pad)
    p_flat = p.reshape(-1)
    bias16 = jnp.full((16,), b[0], jnp.float32)
    out_flat = _sc_gather(p_flat, svt_flat, bias16)
    return out_flat.reshape(B, 1)


# speedup vs baseline: 1.0217x; 1.0217x over previous
import functools

import jax
import jax.numpy as jnp
from jax import lax
from jax.experimental import pallas as pl
from jax.experimental.pallas import tpu as pltpu
from jax.experimental.pallas import tpu_sc as plsc

VOCAB = 100000
D = 128
B = 16384
NSLOT = 5
PCOL = 8

NC = 2
NS = 16
NW = NC * NS
BPW = B // NW
FLAT = BPW * NSLOT
NROW = FLAT // 128
TC_ROWS = 20000

_PERM = {}
_MASK = {}
for _j in range(NSLOT):
    for _k in range(NSLOT):
        lanes = [5 * _i + _j - 16 * _k for _i in range(16)]
        _PERM[(_j, _k)] = [min(15, max(0, _l)) for _l in lanes]
        _MASK[(_j, _k)] = [1.0 if 0 <= _l < 16 else 0.0 for _l in lanes]


def _tc_project_body(g_ref, w_ref, p_ref):
    p_ref[...] = jnp.dot(
        g_ref[...].astype(jnp.bfloat16),
        w_ref[...].astype(jnp.bfloat16),
        preferred_element_type=jnp.float32,
    )


def _tc_project(g_feats, w_pad):
    grid = VOCAB // TC_ROWS
    return pl.pallas_call(
        _tc_project_body,
        grid=(grid,),
        in_specs=[
            pl.BlockSpec((TC_ROWS, D), lambda i: (i, 0)),
            pl.BlockSpec((D, PCOL), lambda i: (0, 0)),
        ],
        out_specs=pl.BlockSpec((TC_ROWS, PCOL), lambda i: (i, 0)),
        out_shape=jax.ShapeDtypeStruct((VOCAB, PCOL), jnp.float32),
    )(g_feats, w_pad)


def _sc_gather(p_flat, samples_flat, bias16):
    mesh = plsc.VectorSubcoreMesh(core_axis_name="c", subcore_axis_name="s")

    @functools.partial(
        pl.kernel,
        mesh=mesh,
        out_type=jax.ShapeDtypeStruct((B,), jnp.float32),
        scratch_types=[
            pltpu.VMEM((FLAT,), jnp.int32),
            pltpu.VMEM((NROW, 128), jnp.int32),
            pltpu.VMEM((FLAT,), jnp.float32),
            pltpu.VMEM((BPW,), jnp.float32),
            pltpu.VMEM((16,), jnp.float32),
            pltpu.SemaphoreType.DMA,
        ],
    )
    def sc_k(pflat_hbm, sflat_hbm, bias_hbm, out_hbm, sv, fidx, gbuf, acc, bv, sem):
        wid = lax.axis_index("s") * NC + lax.axis_index("c")
        base = wid * BPW
        pltpu.sync_copy(sflat_hbm.at[pl.ds(base * NSLOT, FLAT)], sv)
        pltpu.sync_copy(bias_hbm, bv)
        io = lax.iota(jnp.int32, 16)
        handles = []
        for r in range(NROW):
            for c in range(8):
                t = r * 8 + c
                ids = sv[pl.ds(t * 16, 16)]
                slot = lax.rem(io + (t * 16) % NSLOT, NSLOT)
                fidx[r, pl.ds(c * 16, 16)] = ids * PCOL + slot
            handles.append(
                pltpu.async_copy(
                    pflat_hbm.at[fidx.at[r]], gbuf.at[pl.ds(r * 128, 128)], sem
                )
            )
        for h in handles:
            h.wait()
        bias_v = bv[...]
        io5 = io * NSLOT
        perm = {}
        mask = {}
        for j in range(NSLOT):
            for k in range(NSLOT):
                x = io5 + (j - 16 * k)
                perm[(j, k)] = jnp.minimum(jnp.maximum(x, 0), 15)
                mask[(j, k)] = (x >= 0) & (x < 16)
        for c in range(BPW // 16):
            v = [gbuf[pl.ds(c * 80 + 16 * k, 16)] for k in range(NSLOT)]
            tot = bias_v
            for j in range(NSLOT):
                for k in range(NSLOT):
                    tot = tot + jnp.where(mask[(j, k)], v[k][perm[(j, k)]], 0.0)
            acc[pl.ds(c * 16, 16)] = tot
        pltpu.sync_copy(acc, out_hbm.at[pl.ds(base, BPW)])

    return sc_k(p_flat, samples_flat, bias16)


def kernel(g_feats, samples, W, b):
    w_pad = jnp.zeros((D, PCOL), jnp.float32).at[:, :NSLOT].set(
        W.reshape(NSLOT, D).T
    )
    p = _tc_project(g_feats, w_pad)
    p_flat = p.reshape(-1)
    samples_flat = samples.reshape(-1)
    bias16 = jnp.full((16,), b[0], jnp.float32)
    out_flat = _sc_gather(p_flat, samples_flat, bias16)
    return out_flat.reshape(B, 1)


# speedup vs baseline: 1.1101x; 1.0865x over previous
import functools

import jax
import jax.numpy as jnp
from jax import lax
from jax.experimental import pallas as pl
from jax.experimental.pallas import tpu as pltpu
from jax.experimental.pallas import tpu_sc as plsc

VOCAB = 100000
D = 128
B = 16384
NSLOT = 5
IN_SIZE = NSLOT * D

NC = 2
NS = 16
NW = NC * NS
BPW = B // NW
SPC = 16
RPC = SPC * NSLOT
NCHUNK = BPW // SPC
NQ = D // 16


def _sc_kernel(g_feats, samples_flat, w_flat, bias16):
    mesh = plsc.VectorSubcoreMesh(core_axis_name="c", subcore_axis_name="s")

    @functools.partial(
        pl.kernel,
        mesh=mesh,
        out_type=jax.ShapeDtypeStruct((B,), jnp.float32),
        scratch_types=[
            pltpu.VMEM((BPW * NSLOT,), jnp.int32),
            pltpu.VMEM((2, RPC, D), jnp.float32),
            pltpu.VMEM((IN_SIZE,), jnp.float32),
            pltpu.VMEM((16,), jnp.float32),
            pltpu.VMEM((BPW,), jnp.float32),
            pltpu.SemaphoreType.DMA,
            pltpu.SemaphoreType.DMA,
        ],
    )
    def sc_k(g_hbm, sflat_hbm, w_hbm, bias_hbm, out_hbm,
             sv, dbuf, wv, bv, acc, sem0, sem1):
        wid = lax.axis_index("s") * NC + lax.axis_index("c")
        base = wid * BPW
        pltpu.sync_copy(sflat_hbm.at[pl.ds(base * NSLOT, BPW * NSLOT)], sv)
        pltpu.sync_copy(w_hbm, wv)
        pltpu.sync_copy(bias_hbm, bv)
        sems = (sem0, sem1)
        wreg = [[wv[pl.ds(j * D + q * 16, 16)] for q in range(NQ)]
                for j in range(NSLOT)]
        io = lax.iota(jnp.int32, 16)
        bias_v = bv[...]

        def fire(t, b):
            return pltpu.async_copy(
                g_hbm.at[sv.at[pl.ds(t * RPC, RPC)]], dbuf.at[b], sems[b]
            )

        def wait(b):
            pltpu.make_async_copy(
                g_hbm.at[pl.ds(0, RPC), :], dbuf.at[b], sems[b]
            ).wait()

        def compute(t, b):
            res = bias_v
            for i in range(SPC):
                s = None
                for j in range(NSLOT):
                    r = NSLOT * i + j
                    for q in range(NQ):
                        v = dbuf[b, r, pl.ds(q * 16, 16)]
                        term = v * wreg[j][q]
                        s = term if s is None else s + term
                for sh in (1, 2, 4, 8):
                    s = s + s[jnp.bitwise_xor(io, sh)]
                res = jnp.where(io == i, res + s, res)
            acc[pl.ds(t * 16, 16)] = res

        fire(0, 0)
        fire(1, 1)

        def body(it, carry):
            for b in range(2):
                t = it * 2 + b
                wait(b)
                compute(t, b)

                @pl.when(t + 2 < NCHUNK)
                def _():
                    fire(t + 2, b)

            return carry

        lax.fori_loop(0, NCHUNK // 2, body, 0)
        pltpu.sync_copy(acc, out_hbm.at[pl.ds(base, BPW)])

    return sc_k(g_feats, samples_flat, w_flat, bias16)


def kernel(g_feats, samples, W, b):
    samples_flat = samples.reshape(-1)
    w_flat = W.reshape(-1)
    bias16 = jnp.full((16,), b[0], jnp.float32)
    out_flat = _sc_kernel(g_feats, samples_flat, w_flat, bias16)
    return out_flat.reshape(B, 1)
